# baseline (speedup 1.0000x reference)
import functools

import jax
import jax.numpy as jnp
from jax import lax
from jax.experimental import pallas as pl
from jax.experimental.pallas import tpu as pltpu
from jax.experimental.pallas import tpu_sc as plsc

B, S, C, T = 16, 4, 2, 131072
P = B * S
PPW = P // 2


def _flip_body(y_hbm, left_hbm, out_hbm, flags, bufa, bufb, sa, sb, wa, wb):
    cid = lax.axis_index("c")
    p0 = cid * PPW

    pltpu.sync_copy(left_hbm, flags)

    for k in range(0, PPW):
        p = p0 + k
        b = p // S
        s = p % S
        l = flags[p]
        ra = pltpu.async_copy(y_hbm.at[b, s, 0], bufa, sa)
        rb = pltpu.async_copy(y_hbm.at[b, s, 1], bufb, sb)
        ra.wait()
        wda = pltpu.async_copy(bufa, out_hbm.at[b, s, l], wa)
        rb.wait()
        wdb = pltpu.async_copy(bufb, out_hbm.at[b, s, 1 - l], wb)
        wda.wait()
        wdb.wait()


@jax.jit
def _flip(y, lf):
    mesh = plsc.ScalarSubcoreMesh(axis_name="c", num_cores=2)
    return pl.kernel(
        _flip_body,
        out_type=jax.ShapeDtypeStruct((B, S, C, T), jnp.float32),
        mesh=mesh,
        scratch_types=[
            pltpu.SMEM((P,), jnp.int32),
            pltpu.VMEM_SHARED((T,), jnp.float32),
            pltpu.VMEM_SHARED((T,), jnp.float32),
            pltpu.SemaphoreType.DMA,
            pltpu.SemaphoreType.DMA,
            pltpu.SemaphoreType.DMA,
            pltpu.SemaphoreType.DMA,
        ],
    )(y, lf)


def kernel(y, left):
    lf = left.reshape(P).astype(jnp.int32)
    return _flip(y, lf)


# speedup vs baseline: 2.0682x; 2.0682x over previous
import functools

import jax
import jax.numpy as jnp
from jax import lax
from jax.experimental import pallas as pl
from jax.experimental.pallas import tpu as pltpu
from jax.experimental.pallas import tpu_sc as plsc
from jax._src.pallas import mpmd

B, S, C, T = 16, 4, 2, 131072
P = B * S
NW = 32
CHB = 32768
NCH = T // CHB
NB = 3
RPW = 3
TEC_ROWS = NW * RPW
SCS_PAIRS = (C * P - TEC_ROWS) // C // 2


def _tec_body(y_hbm, left_hbm, out_hbm, bufa, bufb):
    def inner(left_v, b0, b1, b2, rs0, rs1, rs2, ws0, ws1, ws2, fsem):
        bufs = [b0, b1, b2]
        rsems = [rs0, rs1, rs2]
        wsems = [ws0, ws1, ws2]

        cid = lax.axis_index("c")
        sid = lax.axis_index("s")
        w = sid * 2 + cid
        r0 = RPW * w

        rows = [r0, r0 + 1, r0 + 2]
        prs = [r // C for r in rows]
        pcs = [r % C for r in rows]
        xfers = [
            (prs[k] // S, prs[k] % S, pcs[k], j * CHB, k)
            for k in range(RPW)
            for j in range(NCH)
        ]
        n = len(xfers)

        def read(i):
            bb, ss, sc, col, _ = xfers[i]
            return pltpu.async_copy(
                y_hbm.at[bb, ss, sc, pl.ds(col, CHB)],
                bufs[i % NB], rsems[i % NB],
            )

        fdesc = pltpu.async_copy(left_hbm, left_v.at[pl.ds(0, P)], fsem)
        rdesc = [None] * NB
        wdesc = [None] * NB
        for t in range(NB - 1):
            rdesc[t] = read(t)
        fdesc.wait()
        p_lo = prs[0]
        lv = left_v[pl.ds(p_lo, 16)]
        f0 = lv[0]
        f1 = lv[1]
        dch = []
        for k in range(RPW):
            o = prs[k] - p_lo
            fk = f0 * (1 - o) + f1 * o
            dch.append(pcs[k] + fk - 2 * pcs[k] * fk)

        for t in range(n):
            rdesc[t % NB].wait()
            bb, ss, _, col, k = xfers[t]
            wdesc[t % NB] = pltpu.async_copy(
                bufs[t % NB], out_hbm.at[bb, ss, dch[k], pl.ds(col, CHB)],
                wsems[t % NB],
            )
            nxt = t + NB - 1
            if nxt < n:
                if t >= 1:
                    wdesc[(t - 1) % NB].wait()
                rdesc[nxt % NB] = read(nxt)
        for j in range(n - NB, n):
            wdesc[j % NB].wait()

    pl.run_scoped(
        inner,
        pltpu.VMEM((P + 16,), jnp.int32),
        *[pltpu.VMEM((CHB,), jnp.float32) for _ in range(NB)],
        *[pltpu.SemaphoreType.DMA for _ in range(2 * NB + 1)],
    )


def _scs_body(y_hbm, left_hbm, out_hbm, bufa, bufb):
    def inner(flags, sa, sb, wa, wb):
        cid = lax.axis_index("c")
        p0 = TEC_ROWS // C + cid * SCS_PAIRS

        pltpu.sync_copy(left_hbm, flags)

        for k in range(SCS_PAIRS):
            p = p0 + k
            bb = p // S
            ss = p % S
            l = flags[p]
            ra = pltpu.async_copy(y_hbm.at[bb, ss, 0], bufa, sa)
            rb = pltpu.async_copy(y_hbm.at[bb, ss, 1], bufb, sb)
            ra.wait()
            wda = pltpu.async_copy(bufa, out_hbm.at[bb, ss, l], wa)
            rb.wait()
            wdb = pltpu.async_copy(bufb, out_hbm.at[bb, ss, 1 - l], wb)
            wda.wait()
            wdb.wait()

    pl.run_scoped(
        inner,
        pltpu.SMEM((P,), jnp.int32),
        *[pltpu.SemaphoreType.DMA for _ in range(4)],
    )


@jax.jit
def _flip(y, lf):
    scalar_mesh = plsc.ScalarSubcoreMesh(axis_name="c", num_cores=2)
    vector_mesh = plsc.VectorSubcoreMesh(
        core_axis_name="c", subcore_axis_name="s"
    )
    return mpmd.mpmd_map(
        [(vector_mesh, _tec_body), (scalar_mesh, _scs_body)],
        out_types=jax.ShapeDtypeStruct((B, S, C, T), jnp.float32),
        scratch_types=[
            pltpu.VMEM_SHARED((T,), jnp.float32),
            pltpu.VMEM_SHARED((T,), jnp.float32),
        ],
    )(y, lf)


def kernel(y, left):
    lf = left.reshape(P).astype(jnp.int32)
    return _flip(y, lf)
